# baseline (speedup 1.0000x reference)
import functools

import jax
import jax.numpy as jnp
from jax import lax
from jax.experimental import pallas as pl
from jax.experimental.pallas import tpu as pltpu
from jax.experimental.pallas import tpu_sc as plsc

N_LEVELS = 2
T = 2 ** 22
MASK = T - 1
BASE_RES = 64
P2 = -1640531535
P3 = 805459861
NUM_POINTS = 1048576

NC = 2
NS = 16
NW = NC * NS
PW = NUM_POINTS // NW
C = 256
NCHUNK = PW // C
NV = C // 16

PROWS = N_LEVELS * T // 4
RPC = 1024
RPT = PROWS // NS


def _sc_forward(meanT, tabp):
    mesh = plsc.VectorSubcoreMesh(core_axis_name="c", subcore_axis_name="s")

    @functools.partial(
        pl.kernel,
        out_type=[jax.ShapeDtypeStruct((4, NUM_POINTS), jnp.float32),
                  jax.ShapeDtypeStruct((PROWS, 8), jnp.float32)],
        mesh=mesh,
        compiler_params=pltpu.CompilerParams(use_tc_tiling_on_sc=False,
                                             needs_layout_passes=False),
        scratch_types=[
            pltpu.VMEM((6, C), jnp.float32),
            pltpu.VMEM((2, 16 * C), jnp.int32),
            pltpu.VMEM((2, 16 * C), jnp.int32),
            pltpu.VMEM((2, 16 * C), jnp.float32),
            pltpu.VMEM((2, 16 * C, 8), jnp.float32),
            pltpu.VMEM((4, C), jnp.float32),
            pltpu.SemaphoreType.DMA,
            pltpu.SemaphoreType.DMA,
        ],
    )
    def k(m_hbm, tab_hbm, out_hbm, p_hbm,
          xyz_v, idx_buf, col_buf, w_buf, rows_v, out_buf, sem0, sem1):
        cid = lax.axis_index("c")
        sid = lax.axis_index("s")
        wid = sid * NC + cid
        iota = lax.broadcasted_iota(jnp.int32, (16,), 0)
        sems = (sem0, sem1)

        rep_in = rows_v.at[0]
        rep_out = rows_v.at[1]
        rvec = lax.shift_right_logical(iota, 3)
        lvec = iota & 7
        rA = lax.shift_right_logical(iota, 2)
        lA = (2 * iota) & 7

        def rep_round(r, carry):
            base = sid * RPT + r * RPC
            pltpu.sync_copy(tab_hbm.at[pl.ds(base, RPC)],
                            rep_in.at[pl.ds(0, RPC)])

            def rep_block(bk, c2):
                b32 = bk * 32
                for m in range(8):
                    va = plsc.load_gather(rep_in,
                                          [rvec + (b32 + 2 * m), lvec])
                    vb = plsc.load_gather(rep_in,
                                          [rvec + (b32 + 16 + 2 * m), lvec])
                    dr = rA + (b32 + 4 * m)
                    plsc.store_scatter(rep_out, [dr, lA], va)
                    plsc.store_scatter(rep_out, [dr, lA + 1], vb)
                return c2

            lax.fori_loop(0, RPC // 32, rep_block, 0)
            pltpu.sync_copy(rep_out.at[pl.ds(0, RPC)],
                            p_hbm.at[pl.ds(base, RPC)])
            return carry

        lax.fori_loop(0, RPT // RPC, rep_round, 0)
        plsc.subcore_barrier()

        def produce(ci, p):
            base = wid * PW + ci * C
            pltpu.sync_copy(m_hbm.at[:, pl.ds(base, C)],
                            xyz_v.at[pl.ds(3 * p, 3)])

            def pass_a(i, c2):
                s = i * 16
                xr = xyz_v[3 * p + 0, pl.ds(s, 16)]
                yr = xyz_v[3 * p + 1, pl.ds(s, 16)]
                zr = xyz_v[3 * p + 2, pl.ds(s, 16)]
                for l in range(N_LEVELS):
                    res = float(BASE_RES * (4 ** l))
                    px = xr * res
                    py = yr * res
                    pz = zr * res
                    ix = px.astype(jnp.int32)
                    iy = py.astype(jnp.int32)
                    iz = pz.astype(jnp.int32)
                    fx = px - ix.astype(jnp.float32)
                    fy = py - iy.astype(jnp.float32)
                    fz = pz - iz.astype(jnp.float32)
                    hx0 = ix
                    hx1 = ix + 1
                    hy0 = iy * P2
                    hy1 = hy0 + P2
                    hz0 = iz * P3
                    hz1 = hz0 + P3
                    wx0 = 1.0 - fx
                    wy0 = 1.0 - fy
                    wz0 = 1.0 - fz
                    for corner in range(8):
                        dx = corner & 1
                        dy = (corner >> 1) & 1
                        dz = (corner >> 2) & 1
                        h = (((hx1 if dx else hx0)
                              ^ (hy1 if dy else hy0)
                              ^ (hz1 if dz else hz0)) & MASK)
                        r0 = lax.shift_right_logical(h, 2) + (l << 20)
                        wcv = (((fx if dx else wx0)
                                * (fy if dy else wy0))
                               * (fz if dz else wz0))
                        b = l * 8 + corner
                        off = b * C + s
                        idx_buf[p, pl.ds(off, 16)] = r0
                        col_buf[p, pl.ds(off, 16)] = lax.shift_left(h & 3, 1)
                        w_buf[p, pl.ds(off, 16)] = wcv
                return c2

            lax.fori_loop(0, NV, pass_a, 0)
            pltpu.async_copy(p_hbm.at[idx_buf.at[p]], rows_v.at[p], sems[p])

        def consume(ci, p):
            base = wid * PW + ci * C
            pltpu.make_async_copy(p_hbm.at[idx_buf.at[p]],
                                  rows_v.at[p], sems[p]).wait()

            def pass_b(i, c2):
                s = i * 16
                rowb = s + iota
                rvp = rows_v.at[p]
                acc = [None] * 4
                for l in range(N_LEVELS):
                    for corner in range(8):
                        b = l * 8 + corner
                        off = b * C + s
                        wc = w_buf[p, pl.ds(off, 16)]
                        colv = col_buf[p, pl.ds(off, 16)]
                        g0 = plsc.load_gather(rvp, [rowb + b * C, colv])
                        g1 = plsc.load_gather(rvp, [rowb + b * C, colv + 1])
                        t0 = wc * g0
                        t1 = wc * g1
                        if corner == 0:
                            acc[2 * l] = t0
                            acc[2 * l + 1] = t1
                        else:
                            acc[2 * l] = acc[2 * l] + t0
                            acc[2 * l + 1] = acc[2 * l + 1] + t1
                for j in range(4):
                    out_buf[j, pl.ds(s, 16)] = jnp.maximum(acc[j], 0.0)
                return c2

            lax.fori_loop(0, NV, pass_b, 0)
            pltpu.sync_copy(out_buf, out_hbm.at[:, pl.ds(base, C)])

        produce(0, 0)

        def pair_body(j, carry):
            i0 = 2 * j
            i1 = i0 + 1
            produce(i1, 1)
            consume(i0, 0)

            @pl.when(i1 + 1 < NCHUNK)
            def _():
                produce(i1 + 1, 0)

            consume(i1, 1)
            return carry

        lax.fori_loop(0, NCHUNK // 2, pair_body, 0)

    return k(meanT, tabp)


def kernel(mean, deformation_codes, decayscales, table):
    del deformation_codes, decayscales
    meanT = mean.T
    tabp = (table.reshape(N_LEVELS, T // 128, 128, 2)
            .transpose(0, 1, 3, 2)
            .reshape(N_LEVELS * T * 2 // 8, 8))
    out, _ = _sc_forward(meanT, tabp)
    return out.T


# speedup vs baseline: 1.1990x; 1.1990x over previous
import functools

import jax
import jax.numpy as jnp
from jax import lax
from jax.experimental import pallas as pl
from jax.experimental.pallas import tpu as pltpu
from jax.experimental.pallas import tpu_sc as plsc

N_LEVELS = 2
T = 2 ** 22
MASK = T - 1
BASE_RES = 64
P2 = -1640531535
P3 = 805459861
NUM_POINTS = 1048576

NC = 2
NS = 16
NW = NC * NS
PW = NUM_POINTS // NW
C = 256
NCHUNK = PW // C
NV = C // 16

PROWS = N_LEVELS * T // 4
RPC = 2048
RPT = PROWS // NS
NR = RPT // RPC


def _sc_forward(meanT, tabp):
    mesh = plsc.VectorSubcoreMesh(core_axis_name="c", subcore_axis_name="s")

    @functools.partial(
        pl.kernel,
        out_type=[jax.ShapeDtypeStruct((4, NUM_POINTS), jnp.float32),
                  jax.ShapeDtypeStruct((PROWS, 8), jnp.float32)],
        mesh=mesh,
        compiler_params=pltpu.CompilerParams(use_tc_tiling_on_sc=False,
                                             needs_layout_passes=False),
        scratch_types=[
            pltpu.VMEM((6, C), jnp.float32),
            pltpu.VMEM((2, 16 * C), jnp.int32),
            pltpu.VMEM((2, 16 * C), jnp.int32),
            pltpu.VMEM((2, 16 * C), jnp.float32),
            pltpu.VMEM((2, 16 * C, 8), jnp.float32),
            pltpu.VMEM((4, C), jnp.float32),
            pltpu.SemaphoreType.DMA,
            pltpu.SemaphoreType.DMA,
            pltpu.SemaphoreType.DMA,
            pltpu.SemaphoreType.DMA,
        ],
    )
    def k(m_hbm, tab_hbm, out_hbm, p_hbm,
          xyz_v, idx_buf, col_buf, w_buf, rows_v, out_buf,
          sem0, sem1, sem2, sem3):
        cid = lax.axis_index("c")
        sid = lax.axis_index("s")
        wid = sid * NC + cid
        iota = lax.broadcasted_iota(jnp.int32, (16,), 0)
        sems = (sem0, sem1)

        rins = (rows_v.at[0, pl.ds(0, RPC)], rows_v.at[1, pl.ds(0, RPC)])
        routs = (rows_v.at[0, pl.ds(RPC, RPC)], rows_v.at[1, pl.ds(RPC, RPC)])
        isems = (sem0, sem1)
        osems = (sem2, sem3)
        rvec = lax.shift_right_logical(iota, 3)
        lvec = iota & 7
        rA = lax.shift_right_logical(iota, 2)
        lA = (2 * iota) & 7

        def rbase(r):
            return sid * RPT + r * RPC

        def rep_stage(r, p):
            @pl.when(r + 1 < NR)
            def _():
                pltpu.async_copy(tab_hbm.at[pl.ds(rbase(r + 1), RPC)],
                                 rins[1 - p], isems[1 - p])

            pltpu.make_async_copy(tab_hbm.at[pl.ds(rbase(r), RPC)],
                                  rins[p], isems[p]).wait()

            @pl.when(r >= 2)
            def _():
                pltpu.make_async_copy(routs[p],
                                      p_hbm.at[pl.ds(rbase(r - 2), RPC)],
                                      osems[p]).wait()

            def rep_block(bk, c2):
                b32 = bk * 32
                for m in range(8):
                    va = plsc.load_gather(rins[p],
                                          [rvec + (b32 + 2 * m), lvec])
                    vb = plsc.load_gather(rins[p],
                                          [rvec + (b32 + 16 + 2 * m), lvec])
                    dr = rA + (b32 + 4 * m)
                    plsc.store_scatter(routs[p], [dr, lA], va)
                    plsc.store_scatter(routs[p], [dr, lA + 1], vb)
                return c2

            lax.fori_loop(0, RPC // 32, rep_block, 0)
            pltpu.async_copy(routs[p], p_hbm.at[pl.ds(rbase(r), RPC)],
                             osems[p])

        pltpu.async_copy(tab_hbm.at[pl.ds(rbase(0), RPC)], rins[0], isems[0])

        def rep_pair(j, carry):
            rep_stage(2 * j, 0)
            rep_stage(2 * j + 1, 1)
            return carry

        lax.fori_loop(0, NR // 2, rep_pair, 0)
        pltpu.make_async_copy(routs[0], p_hbm.at[pl.ds(rbase(NR - 2), RPC)],
                              osems[0]).wait()
        pltpu.make_async_copy(routs[1], p_hbm.at[pl.ds(rbase(NR - 1), RPC)],
                              osems[1]).wait()
        plsc.subcore_barrier()

        def produce(ci, p):
            base = wid * PW + ci * C
            pltpu.sync_copy(m_hbm.at[:, pl.ds(base, C)],
                            xyz_v.at[pl.ds(3 * p, 3)])

            def pass_a(i, c2):
                s = i * 16
                xr = xyz_v[3 * p + 0, pl.ds(s, 16)]
                yr = xyz_v[3 * p + 1, pl.ds(s, 16)]
                zr = xyz_v[3 * p + 2, pl.ds(s, 16)]
                for l in range(N_LEVELS):
                    res = float(BASE_RES * (4 ** l))
                    px = xr * res
                    py = yr * res
                    pz = zr * res
                    ix = px.astype(jnp.int32)
                    iy = py.astype(jnp.int32)
                    iz = pz.astype(jnp.int32)
                    fx = px - ix.astype(jnp.float32)
                    fy = py - iy.astype(jnp.float32)
                    fz = pz - iz.astype(jnp.float32)
                    hx0 = ix
                    hx1 = ix + 1
                    hy0 = iy * P2
                    hy1 = hy0 + P2
                    hz0 = iz * P3
                    hz1 = hz0 + P3
                    wx0 = 1.0 - fx
                    wy0 = 1.0 - fy
                    wz0 = 1.0 - fz
                    for corner in range(8):
                        dx = corner & 1
                        dy = (corner >> 1) & 1
                        dz = (corner >> 2) & 1
                        h = (((hx1 if dx else hx0)
                              ^ (hy1 if dy else hy0)
                              ^ (hz1 if dz else hz0)) & MASK)
                        r0 = lax.shift_right_logical(h, 2) + (l << 20)
                        wcv = (((fx if dx else wx0)
                                * (fy if dy else wy0))
                               * (fz if dz else wz0))
                        b = l * 8 + corner
                        off = b * C + s
                        idx_buf[p, pl.ds(off, 16)] = r0
                        col_buf[p, pl.ds(off, 16)] = lax.shift_left(h & 3, 1)
                        w_buf[p, pl.ds(off, 16)] = wcv
                return c2

            lax.fori_loop(0, NV, pass_a, 0)
            pltpu.async_copy(p_hbm.at[idx_buf.at[p]], rows_v.at[p], sems[p])

        def consume(ci, p):
            base = wid * PW + ci * C
            pltpu.make_async_copy(p_hbm.at[idx_buf.at[p]],
                                  rows_v.at[p], sems[p]).wait()

            def pass_b(i, c2):
                s = i * 16
                rowb = s + iota
                rvp = rows_v.at[p]
                acc = [None] * 4
                for l in range(N_LEVELS):
                    for corner in range(8):
                        b = l * 8 + corner
                        off = b * C + s
                        wc = w_buf[p, pl.ds(off, 16)]
                        colv = col_buf[p, pl.ds(off, 16)]
                        g0 = plsc.load_gather(rvp, [rowb + b * C, colv])
                        g1 = plsc.load_gather(rvp, [rowb + b * C, colv + 1])
                        t0 = wc * g0
                        t1 = wc * g1
                        if corner == 0:
                            acc[2 * l] = t0
                            acc[2 * l + 1] = t1
                        else:
                            acc[2 * l] = acc[2 * l] + t0
                            acc[2 * l + 1] = acc[2 * l + 1] + t1
                for j in range(4):
                    out_buf[j, pl.ds(s, 16)] = jnp.maximum(acc[j], 0.0)
                return c2

            lax.fori_loop(0, NV, pass_b, 0)
            pltpu.sync_copy(out_buf, out_hbm.at[:, pl.ds(base, C)])

        produce(0, 0)

        def pair_body(j, carry):
            i0 = 2 * j
            i1 = i0 + 1
            produce(i1, 1)
            consume(i0, 0)

            @pl.when(i1 + 1 < NCHUNK)
            def _():
                produce(i1 + 1, 0)

            consume(i1, 1)
            return carry

        lax.fori_loop(0, NCHUNK // 2, pair_body, 0)

    return k(meanT, tabp)


def kernel(mean, deformation_codes, decayscales, table):
    del deformation_codes, decayscales
    meanT = mean.T
    tabp = (table.reshape(N_LEVELS, T // 128, 128, 2)
            .transpose(0, 1, 3, 2)
            .reshape(N_LEVELS * T * 2 // 8, 8))
    out, _ = _sc_forward(meanT, tabp)
    return out.T


# speedup vs baseline: 1.5294x; 1.2755x over previous
import functools

import jax
import jax.numpy as jnp
from jax import lax
from jax.experimental import pallas as pl
from jax.experimental.pallas import tpu as pltpu
from jax.experimental.pallas import tpu_sc as plsc

N_LEVELS = 2
T = 2 ** 22
MASK = T - 1
BASE_RES = 64
P2 = -1640531535
P3 = 805459861
NUM_POINTS = 1048576

NC = 2
NS = 16
NW = NC * NS
PW = NUM_POINTS // NW
C = 256
NCHUNK = PW // C
NV = C // 16

PROWS = N_LEVELS * T // 4
RPC = 2048
RPT = PROWS // NS
NR = RPT // RPC


def _sc_forward(meanT, tabp):
    mesh = plsc.VectorSubcoreMesh(core_axis_name="c", subcore_axis_name="s")

    @functools.partial(
        pl.kernel,
        out_type=[jax.ShapeDtypeStruct((4, NUM_POINTS), jnp.float32),
                  jax.ShapeDtypeStruct((PROWS, 8), jnp.float32)],
        mesh=mesh,
        compiler_params=pltpu.CompilerParams(use_tc_tiling_on_sc=False,
                                             needs_layout_passes=False),
        scratch_types=[
            pltpu.VMEM((6, C), jnp.float32),
            pltpu.VMEM((2, 16 * C), jnp.int32),
            pltpu.VMEM((2, 16 * C), jnp.int32),
            pltpu.VMEM((2, 16 * C), jnp.float32),
            pltpu.VMEM((2, 8 * C), jnp.int32),
            pltpu.VMEM((2, 16 * C, 8), jnp.float32),
            pltpu.VMEM((4, C), jnp.float32),
            pltpu.SemaphoreType.DMA,
            pltpu.SemaphoreType.DMA,
            pltpu.SemaphoreType.DMA,
            pltpu.SemaphoreType.DMA,
        ],
    )
    def k(m_hbm, tab_hbm, out_hbm, p_hbm,
          xyz_v, idx_buf, col_buf, w_buf, rsel_buf, rows_v, out_buf,
          sem0, sem1, sem2, sem3):
        cid = lax.axis_index("c")
        sid = lax.axis_index("s")
        wid = sid * NC + cid
        iota = lax.broadcasted_iota(jnp.int32, (16,), 0)
        sems = (sem0, sem1)

        rins = (rows_v.at[0, pl.ds(0, RPC)], rows_v.at[1, pl.ds(0, RPC)])
        routs = (rows_v.at[0, pl.ds(RPC, RPC)], rows_v.at[1, pl.ds(RPC, RPC)])
        isems = (sem0, sem1)
        osems = (sem2, sem3)
        rvec = lax.shift_right_logical(iota, 3)
        lvec = iota & 7
        rA = lax.shift_right_logical(iota, 2)
        lA = (2 * iota) & 7

        def rbase(r):
            return sid * RPT + r * RPC

        def rep_stage(r, p):
            @pl.when(r + 1 < NR)
            def _():
                pltpu.async_copy(tab_hbm.at[pl.ds(rbase(r + 1), RPC)],
                                 rins[1 - p], isems[1 - p])

            pltpu.make_async_copy(tab_hbm.at[pl.ds(rbase(r), RPC)],
                                  rins[p], isems[p]).wait()

            @pl.when(r >= 2)
            def _():
                pltpu.make_async_copy(routs[p],
                                      p_hbm.at[pl.ds(rbase(r - 2), RPC)],
                                      osems[p]).wait()

            def rep_block(bk, c2):
                b32 = bk * 32
                for m in range(8):
                    va = plsc.load_gather(rins[p],
                                          [rvec + (b32 + 2 * m), lvec])
                    vb = plsc.load_gather(rins[p],
                                          [rvec + (b32 + 16 + 2 * m), lvec])
                    dr = rA + (b32 + 4 * m)
                    plsc.store_scatter(routs[p], [dr, lA], va)
                    plsc.store_scatter(routs[p], [dr, lA + 1], vb)
                return c2

            lax.fori_loop(0, RPC // 32, rep_block, 0)
            pltpu.async_copy(routs[p], p_hbm.at[pl.ds(rbase(r), RPC)],
                             osems[p])

        pltpu.async_copy(tab_hbm.at[pl.ds(rbase(0), RPC)], rins[0], isems[0])

        def rep_pair(j, carry):
            rep_stage(2 * j, 0)
            rep_stage(2 * j + 1, 1)
            return carry

        lax.fori_loop(0, NR // 2, rep_pair, 0)
        pltpu.make_async_copy(routs[0], p_hbm.at[pl.ds(rbase(NR - 2), RPC)],
                              osems[0]).wait()
        pltpu.make_async_copy(routs[1], p_hbm.at[pl.ds(rbase(NR - 1), RPC)],
                              osems[1]).wait()
        plsc.subcore_barrier()

        def produce(ci, p):
            base = wid * PW + ci * C
            pltpu.sync_copy(m_hbm.at[:, pl.ds(base, C)],
                            xyz_v.at[pl.ds(3 * p, 3)])

            def pass_a(i, c2):
                s = i * 16
                xr = xyz_v[3 * p + 0, pl.ds(s, 16)]
                yr = xyz_v[3 * p + 1, pl.ds(s, 16)]
                zr = xyz_v[3 * p + 2, pl.ds(s, 16)]
                for l in range(N_LEVELS):
                    res = float(BASE_RES * (4 ** l))
                    px = xr * res
                    py = yr * res
                    pz = zr * res
                    ix = px.astype(jnp.int32)
                    iy = py.astype(jnp.int32)
                    iz = pz.astype(jnp.int32)
                    fx = px - ix.astype(jnp.float32)
                    fy = py - iy.astype(jnp.float32)
                    fz = pz - iz.astype(jnp.float32)
                    hx0 = ix
                    hx1 = ix + 1
                    hy0 = iy * P2
                    hy1 = hy0 + P2
                    hz0 = iz * P3
                    hz1 = hz0 + P3
                    wx0 = 1.0 - fx
                    wy0 = 1.0 - fy
                    wz0 = 1.0 - fz
                    for yz in range(4):
                        dy = yz & 1
                        dz = yz >> 1
                        hyz = (hy1 if dy else hy0) ^ (hz1 if dz else hz0)
                        wyv = fy if dy else wy0
                        wzv = fz if dz else wz0
                        h0 = (hx0 ^ hyz) & MASK
                        h1 = (hx1 ^ hyz) & MASK
                        r0 = lax.shift_right_logical(h0, 2) + (l << 20)
                        r1 = lax.shift_right_logical(h1, 2) + (l << 20)
                        shared = r0 == r1
                        b0 = l * 8 + 2 * yz
                        off0 = b0 * C + s
                        off1 = off0 + C
                        idx_buf[p, pl.ds(off0, 16)] = r0
                        idx_buf[p, pl.ds(off1, 16)] = jnp.where(shared, -1, r1)
                        col_buf[p, pl.ds(off0, 16)] = lax.shift_left(h0 & 3, 1)
                        col_buf[p, pl.ds(off1, 16)] = lax.shift_left(h1 & 3, 1)
                        rsel_buf[p, pl.ds((l * 4 + yz) * C + s, 16)] = (
                            jnp.where(shared, 0, C))
                        w_buf[p, pl.ds(off0, 16)] = (wx0 * wyv) * wzv
                        w_buf[p, pl.ds(off1, 16)] = (fx * wyv) * wzv
                return c2

            lax.fori_loop(0, NV, pass_a, 0)
            pltpu.async_copy(
                p_hbm.at[plsc.Indices(idx_buf.at[p], ignored_value=-1)],
                rows_v.at[p], sems[p])

        def consume(ci, p):
            base = wid * PW + ci * C
            pltpu.make_async_copy(
                p_hbm.at[plsc.Indices(idx_buf.at[p], ignored_value=-1)],
                rows_v.at[p], sems[p]).wait()

            def pass_b(i, c2):
                s = i * 16
                rowb = s + iota
                rvp = rows_v.at[p]
                acc = [None] * 4
                for l in range(N_LEVELS):
                    for corner in range(8):
                        b = l * 8 + corner
                        off = b * C + s
                        wc = w_buf[p, pl.ds(off, 16)]
                        colv = col_buf[p, pl.ds(off, 16)]
                        if corner & 1:
                            rsv = rsel_buf[p, pl.ds((l * 4 + corner // 2) * C
                                                    + s, 16)]
                            rowv = rowb + rsv + (b - 1) * C
                        else:
                            rowv = rowb + b * C
                        g0 = plsc.load_gather(rvp, [rowv, colv])
                        g1 = plsc.load_gather(rvp, [rowv, colv + 1])
                        t0 = wc * g0
                        t1 = wc * g1
                        if corner == 0:
                            acc[2 * l] = t0
                            acc[2 * l + 1] = t1
                        else:
                            acc[2 * l] = acc[2 * l] + t0
                            acc[2 * l + 1] = acc[2 * l + 1] + t1
                for j in range(4):
                    out_buf[j, pl.ds(s, 16)] = jnp.maximum(acc[j], 0.0)
                return c2

            lax.fori_loop(0, NV, pass_b, 0)
            pltpu.sync_copy(out_buf, out_hbm.at[:, pl.ds(base, C)])

        produce(0, 0)

        def pair_body(j, carry):
            i0 = 2 * j
            i1 = i0 + 1
            produce(i1, 1)
            consume(i0, 0)

            @pl.when(i1 + 1 < NCHUNK)
            def _():
                produce(i1 + 1, 0)

            consume(i1, 1)
            return carry

        lax.fori_loop(0, NCHUNK // 2, pair_body, 0)

    return k(meanT, tabp)


def kernel(mean, deformation_codes, decayscales, table):
    del deformation_codes, decayscales
    meanT = mean.T
    tabp = (table.reshape(N_LEVELS, T // 128, 128, 2)
            .transpose(0, 1, 3, 2)
            .reshape(N_LEVELS * T * 2 // 8, 8))
    out, _ = _sc_forward(meanT, tabp)
    return out.T


# speedup vs baseline: 1.5543x; 1.0163x over previous
import functools

import jax
import jax.numpy as jnp
from jax import lax
from jax.experimental import pallas as pl
from jax.experimental.pallas import tpu as pltpu
from jax.experimental.pallas import tpu_sc as plsc

N_LEVELS = 2
T = 2 ** 22
MASK = T - 1
BASE_RES = 64
P2 = -1640531535
P3 = 805459861
NUM_POINTS = 1048576

NC = 2
NS = 16
NW = NC * NS
PW = NUM_POINTS // NW
C = 256
NCHUNK = PW // C
NV = C // 16

PROWS = N_LEVELS * T // 4
RPC = 2048
RPT = PROWS // NS
NR = RPT // RPC


def _sc_forward(meanT, tabp):
    mesh = plsc.VectorSubcoreMesh(core_axis_name="c", subcore_axis_name="s")

    @functools.partial(
        pl.kernel,
        out_type=[jax.ShapeDtypeStruct((4, NUM_POINTS), jnp.float32),
                  jax.ShapeDtypeStruct((PROWS, 8), jnp.float32)],
        mesh=mesh,
        compiler_params=pltpu.CompilerParams(use_tc_tiling_on_sc=False,
                                             needs_layout_passes=False),
        scratch_types=[
            pltpu.VMEM((6, C), jnp.float32),
            pltpu.VMEM((2, 16 * C), jnp.int32),
            pltpu.VMEM((2, 16 * C), jnp.int32),
            pltpu.VMEM((2, 16 * C), jnp.float32),
            pltpu.VMEM((2, 8 * C), jnp.int32),
            pltpu.VMEM((2, 16 * C, 8), jnp.float32),
            pltpu.VMEM((4, C), jnp.float32),
            pltpu.SemaphoreType.DMA,
            pltpu.SemaphoreType.DMA,
            pltpu.SemaphoreType.DMA,
            pltpu.SemaphoreType.DMA,
        ],
    )
    def k(m_hbm, tab_hbm, out_hbm, p_hbm,
          xyz_v, idx_buf, col_buf, w_buf, rsel_buf, rows_v, out_buf,
          sem0, sem1, sem2, sem3):
        cid = lax.axis_index("c")
        sid = lax.axis_index("s")
        wid = sid * NC + cid
        iota = lax.broadcasted_iota(jnp.int32, (16,), 0)
        sems = (sem0, sem1)

        rins = (rows_v.at[0, pl.ds(0, RPC)], rows_v.at[1, pl.ds(0, RPC)])
        routs = (rows_v.at[0, pl.ds(RPC, RPC)], rows_v.at[1, pl.ds(RPC, RPC)])
        isems = (sem0, sem1)
        osems = (sem2, sem3)
        rvec = lax.shift_right_logical(iota, 3)
        lvec = iota & 7
        rA = lax.shift_right_logical(iota, 2)
        lA = (2 * iota) & 7

        def rbase(r):
            return sid * RPT + r * RPC

        def rep_stage(r, p):
            @pl.when(r + 1 < NR)
            def _():
                pltpu.async_copy(tab_hbm.at[pl.ds(rbase(r + 1), RPC)],
                                 rins[1 - p], isems[1 - p])

            pltpu.make_async_copy(tab_hbm.at[pl.ds(rbase(r), RPC)],
                                  rins[p], isems[p]).wait()

            @pl.when(r >= 2)
            def _():
                pltpu.make_async_copy(routs[p],
                                      p_hbm.at[pl.ds(rbase(r - 2), RPC)],
                                      osems[p]).wait()

            @plsc.parallel_loop(0, RPC // 32, 1, unroll=2)
            def rep_block(bk):
                b32 = bk * 32
                for m in range(8):
                    va = plsc.load_gather(rins[p],
                                          [rvec + (b32 + 2 * m), lvec])
                    vb = plsc.load_gather(rins[p],
                                          [rvec + (b32 + 16 + 2 * m), lvec])
                    dr = rA + (b32 + 4 * m)
                    plsc.store_scatter(routs[p], [dr, lA], va)
                    plsc.store_scatter(routs[p], [dr, lA + 1], vb)
            pltpu.async_copy(routs[p], p_hbm.at[pl.ds(rbase(r), RPC)],
                             osems[p])

        pltpu.async_copy(tab_hbm.at[pl.ds(rbase(0), RPC)], rins[0], isems[0])

        def rep_pair(j, carry):
            rep_stage(2 * j, 0)
            rep_stage(2 * j + 1, 1)
            return carry

        lax.fori_loop(0, NR // 2, rep_pair, 0)
        pltpu.make_async_copy(routs[0], p_hbm.at[pl.ds(rbase(NR - 2), RPC)],
                              osems[0]).wait()
        pltpu.make_async_copy(routs[1], p_hbm.at[pl.ds(rbase(NR - 1), RPC)],
                              osems[1]).wait()
        plsc.subcore_barrier()

        def produce(ci, p):
            base = wid * PW + ci * C
            pltpu.sync_copy(m_hbm.at[:, pl.ds(base, C)],
                            xyz_v.at[pl.ds(3 * p, 3)])

            @plsc.parallel_loop(0, NV, 1, unroll=2)
            def pass_a(i):
                s = i * 16
                xr = xyz_v[3 * p + 0, pl.ds(s, 16)]
                yr = xyz_v[3 * p + 1, pl.ds(s, 16)]
                zr = xyz_v[3 * p + 2, pl.ds(s, 16)]
                for l in range(N_LEVELS):
                    res = float(BASE_RES * (4 ** l))
                    px = xr * res
                    py = yr * res
                    pz = zr * res
                    ix = px.astype(jnp.int32)
                    iy = py.astype(jnp.int32)
                    iz = pz.astype(jnp.int32)
                    fx = px - ix.astype(jnp.float32)
                    fy = py - iy.astype(jnp.float32)
                    fz = pz - iz.astype(jnp.float32)
                    hx0 = ix
                    hx1 = ix + 1
                    hy0 = iy * P2
                    hy1 = hy0 + P2
                    hz0 = iz * P3
                    hz1 = hz0 + P3
                    wx0 = 1.0 - fx
                    wy0 = 1.0 - fy
                    wz0 = 1.0 - fz
                    for yz in range(4):
                        dy = yz & 1
                        dz = yz >> 1
                        hyz = (hy1 if dy else hy0) ^ (hz1 if dz else hz0)
                        wyv = fy if dy else wy0
                        wzv = fz if dz else wz0
                        h0 = (hx0 ^ hyz) & MASK
                        h1 = (hx1 ^ hyz) & MASK
                        r0 = lax.shift_right_logical(h0, 2) + (l << 20)
                        r1 = lax.shift_right_logical(h1, 2) + (l << 20)
                        shared = r0 == r1
                        b0 = l * 8 + 2 * yz
                        off0 = b0 * C + s
                        off1 = off0 + C
                        idx_buf[p, pl.ds(off0, 16)] = r0
                        idx_buf[p, pl.ds(off1, 16)] = jnp.where(shared, -1, r1)
                        col_buf[p, pl.ds(off0, 16)] = lax.shift_left(h0 & 3, 1)
                        col_buf[p, pl.ds(off1, 16)] = lax.shift_left(h1 & 3, 1)
                        rsel_buf[p, pl.ds((l * 4 + yz) * C + s, 16)] = (
                            jnp.where(shared, 0, C))
                        w_buf[p, pl.ds(off0, 16)] = (wx0 * wyv) * wzv
                        w_buf[p, pl.ds(off1, 16)] = (fx * wyv) * wzv

            pltpu.async_copy(
                p_hbm.at[plsc.Indices(idx_buf.at[p], ignored_value=-1)],
                rows_v.at[p], sems[p])

        def consume(ci, p):
            base = wid * PW + ci * C
            pltpu.make_async_copy(
                p_hbm.at[plsc.Indices(idx_buf.at[p], ignored_value=-1)],
                rows_v.at[p], sems[p]).wait()

            @plsc.parallel_loop(0, NV, 1, unroll=2)
            def pass_b(i):
                s = i * 16
                rowb = s + iota
                rvp = rows_v.at[p]
                acc = [None] * 4
                for l in range(N_LEVELS):
                    for corner in range(8):
                        b = l * 8 + corner
                        off = b * C + s
                        wc = w_buf[p, pl.ds(off, 16)]
                        colv = col_buf[p, pl.ds(off, 16)]
                        if corner & 1:
                            rsv = rsel_buf[p, pl.ds((l * 4 + corner // 2) * C
                                                    + s, 16)]
                            rowv = rowb + rsv + (b - 1) * C
                        else:
                            rowv = rowb + b * C
                        g0 = plsc.load_gather(rvp, [rowv, colv])
                        g1 = plsc.load_gather(rvp, [rowv, colv + 1])
                        t0 = wc * g0
                        t1 = wc * g1
                        if corner == 0:
                            acc[2 * l] = t0
                            acc[2 * l + 1] = t1
                        else:
                            acc[2 * l] = acc[2 * l] + t0
                            acc[2 * l + 1] = acc[2 * l + 1] + t1
                for j in range(4):
                    out_buf[j, pl.ds(s, 16)] = jnp.maximum(acc[j], 0.0)

            pltpu.sync_copy(out_buf, out_hbm.at[:, pl.ds(base, C)])

        produce(0, 0)

        def pair_body(j, carry):
            i0 = 2 * j
            i1 = i0 + 1
            produce(i1, 1)
            consume(i0, 0)

            @pl.when(i1 + 1 < NCHUNK)
            def _():
                produce(i1 + 1, 0)

            consume(i1, 1)
            return carry

        lax.fori_loop(0, NCHUNK // 2, pair_body, 0)

    return k(meanT, tabp)


def kernel(mean, deformation_codes, decayscales, table):
    del deformation_codes, decayscales
    meanT = mean.T
    tabp = (table.reshape(N_LEVELS, T // 128, 128, 2)
            .transpose(0, 1, 3, 2)
            .reshape(N_LEVELS * T * 2 // 8, 8))
    out, _ = _sc_forward(meanT, tabp)
    return out.T
